# baseline (speedup 1.0000x reference)
import functools

import jax
import jax.numpy as jnp
from jax import lax
from jax.experimental import pallas as pl
from jax.experimental.pallas import tpu as pltpu
from jax.experimental.pallas import tpu_sc as plsc

B, S, D = 16, 4096, 256
NW = 32
ROWS_PER_W = B * S // NW
CHUNK = 128
NCHUNK = ROWS_PER_W // CHUNK
L = 16

_mesh = plsc.VectorSubcoreMesh(core_axis_name="c", subcore_axis_name="s")


@functools.partial(
    pl.kernel,
    mesh=_mesh,
    out_type=jax.ShapeDtypeStruct((B * S, D), jnp.float32),
    scratch_types=[
        pltpu.VMEM((NCHUNK, CHUNK), jnp.int32),
        pltpu.VMEM((CHUNK, D), jnp.float32),
        pltpu.VMEM((CHUNK, D), jnp.float32),
        pltpu.VMEM((CHUNK, D), jnp.float32),
        pltpu.SemaphoreType.DMA,
        pltpu.SemaphoreType.DMA,
        pltpu.SemaphoreType.DMA,
        pltpu.SemaphoreType.DMA,
        pltpu.SemaphoreType.DMA,
        pltpu.SemaphoreType.DMA,
    ],
)
def _zcurve_gather(x_hbm, idx_hbm, out_hbm, gidx_v,
                   rows0_v, rows1_v, rows2_v,
                   gsem0, gsem1, gsem2, ssem0, ssem1, ssem2):
    wid = lax.axis_index("s") * 2 + lax.axis_index("c")
    b = wid // 2
    h = wid % 2
    out_base = wid * ROWS_PER_W
    off = b * S

    pltpu.sync_copy(idx_hbm.at[pl.ds(h * NCHUNK, NCHUNK)], gidx_v)

    NBUF = 3
    bufs = (rows0_v, rows1_v, rows2_v)
    gsems = (gsem0, gsem1, gsem2)
    ssems = (ssem0, ssem1, ssem2)
    g_copies = [None] * NBUF
    s_copies = [None] * NBUF
    for c in range(NCHUNK):
        p = c % NBUF
        for s in range(CHUNK // L):
            gidx_v[c, pl.ds(s * L, L)] = gidx_v[c, pl.ds(s * L, L)] + off
        if c >= NBUF:
            s_copies[p].wait()
        g_copies[p] = pltpu.async_copy(x_hbm.at[gidx_v.at[c]], bufs[p], gsems[p])
        if c > 0:
            q = (c - 1) % NBUF
            g_copies[q].wait()
            s_copies[q] = pltpu.async_copy(
                bufs[q], out_hbm.at[pl.ds(out_base + (c - 1) * CHUNK, CHUNK)],
                ssems[q],
            )
    q = (NCHUNK - 1) % NBUF
    g_copies[q].wait()
    s_copies[q] = pltpu.async_copy(
        bufs[q], out_hbm.at[pl.ds(out_base + (NCHUNK - 1) * CHUNK, CHUNK)],
        ssems[q],
    )
    for c in range(max(0, NCHUNK - NBUF), NCHUNK):
        s_copies[c % NBUF].wait()


def kernel(x, forward_shuffle_idx):
    x2 = x.reshape(B * S, D)
    idx2d = forward_shuffle_idx.reshape(NW, CHUNK)
    out = _zcurve_gather(x2, idx2d)
    return out.reshape(B, S, D)
